# baseline (speedup 1.0000x reference)
import functools

import numpy as np
import jax
import jax.numpy as jnp
from jax import lax
from jax.experimental import pallas as pl
from jax.experimental.pallas import tpu as pltpu, tpu_sc as plsc

_D = 768
_BATCH = 4
_SEQ = 2048
_ROWS = _BATCH * _SEQ

_NW = 32
_PW = _SEQ // _NW
_HC = 32
_NCK = (_BATCH * _PW) // _HC
_VPR = _D // 16


def _sinusoidal_pe(max_len, d_model):
    pos = np.arange(max_len, dtype=np.float32)[:, None]
    div = np.exp(
        np.arange(0, d_model, 2, dtype=np.float32) * (-np.log(10000.0) / d_model)
    )
    pe = np.zeros((max_len, d_model), dtype=np.float32)
    pe[:, 0::2] = np.sin(pos * div)
    pe[:, 1::2] = np.cos(pos * div)
    return jnp.asarray(pe)


_PE = _sinusoidal_pe(_SEQ, _D).reshape(-1)

_mesh = plsc.VectorSubcoreMesh(core_axis_name="c", subcore_axis_name="s")


@functools.partial(
    pl.kernel,
    out_type=jax.ShapeDtypeStruct((_ROWS, _D), jnp.float32),
    mesh=_mesh,
    scratch_types=[
        pltpu.VMEM((_NCK, _HC), jnp.int32),
        pltpu.VMEM((_HC * _D,), jnp.float32),
        pltpu.VMEM((_HC, _D), jnp.float32),
        pltpu.VMEM((_HC, _D), jnp.float32),
        pltpu.VMEM((_HC, _D), jnp.float32),
        pltpu.VMEM((_HC, _D), jnp.float32),
        pltpu.SemaphoreType.DMA,
        pltpu.SemaphoreType.DMA,
        pltpu.SemaphoreType.DMA,
        pltpu.SemaphoreType.DMA,
        pltpu.SemaphoreType.DMA,
        pltpu.SemaphoreType.DMA,
        pltpu.SemaphoreType.DMA,
        pltpu.SemaphoreType.DMA,
        pltpu.SemaphoreType.DMA,
    ],
)
def _emb_kernel(idx_hbm, table_hbm, pe_hbm, out_hbm,
                idx_v, pe_buf, rows0, rows1, rows2, rows3,
                g0, g1, g2, g3, s0, s1, s2, s3, psem):
    wid = lax.axis_index("s") * 2 + lax.axis_index("c")
    pbase = wid * _PW
    bufs = (rows0, rows1, rows2, rows3)
    gsems = (g0, g1, g2, g3)
    ssems = (s0, s1, s2, s3)
    pltpu.sync_copy(idx_hbm.at[wid], idx_v)
    gd = [None] * _NCK
    sd = [None] * _NCK
    gd[0] = pltpu.async_copy(table_hbm.at[idx_v.at[0]], bufs[0], gsems[0])
    pe_d = pltpu.async_copy(pe_hbm.at[pl.ds(pbase * _D, _HC * _D)], pe_buf, psem)
    for k in range(_NCK):
        j = k % 4
        h, b = k // _BATCH, k % _BATCH
        if k + 1 < _NCK:
            jn = (k + 1) % 4
            if k - 3 >= 0:
                sd[k - 3].wait()
            gd[k + 1] = pltpu.async_copy(
                table_hbm.at[idx_v.at[k + 1]], bufs[jn], gsems[jn]
            )
        gd[k].wait()
        if k == 0 or k == _BATCH:
            pe_d.wait()

        @plsc.parallel_loop(0, _HC, unroll=2)
        def add_row(r):
            pe_off = r * _D
            for v in range(_VPR):
                plsc.addupdate(
                    bufs[j].at[r, pl.ds(v * 16, 16)],
                    pe_buf[pl.ds(pe_off + v * 16, 16)],
                )

        if k == _BATCH - 1:
            pe_d = pltpu.async_copy(
                pe_hbm.at[pl.ds((pbase + _HC) * _D, _HC * _D)], pe_buf, psem
            )
        sd[k] = pltpu.async_copy(
            bufs[j], out_hbm.at[pl.ds(b * _SEQ + pbase + h * _HC, _HC)], ssems[j]
        )
    sd[_NCK - 2].wait()
    sd[_NCK - 1].wait()


def _pe_on_device():
    pos = lax.broadcasted_iota(jnp.float32, (_SEQ, _D // 2), 0)
    two_i = lax.broadcasted_iota(jnp.float32, (_SEQ, _D // 2), 1) * 2.0
    ang = pos * jnp.exp(two_i * (-np.log(10000.0) / _D))
    pe = jnp.stack([jnp.sin(ang), jnp.cos(ang)], axis=-1)
    return pe.reshape(-1)


def kernel(x, table):
    idx = (
        x.reshape(_BATCH, _NW, 2, _HC)
        .transpose(1, 2, 0, 3)
        .reshape(_NW, _NCK, _HC)
        .astype(jnp.int32)
    )
    out = _emb_kernel(idx, table, _PE)
    return out.reshape(_BATCH, _SEQ, _D)


# speedup vs baseline: 1.0947x; 1.0947x over previous
import functools

import numpy as np
import jax
import jax.numpy as jnp
from jax import lax
from jax.experimental import pallas as pl
from jax.experimental.pallas import tpu as pltpu, tpu_sc as plsc

_D = 768
_BATCH = 4
_SEQ = 2048
_ROWS = _BATCH * _SEQ

_NW = 32
_PW = _SEQ // _NW
_PC = 8
_HC = _PC * _BATCH
_NCK = _PW // _PC
_VPR = _D // 16


def _sinusoidal_pe(max_len, d_model):
    pos = np.arange(max_len, dtype=np.float32)[:, None]
    div = np.exp(
        np.arange(0, d_model, 2, dtype=np.float32) * (-np.log(10000.0) / d_model)
    )
    pe = np.zeros((max_len, d_model), dtype=np.float32)
    pe[:, 0::2] = np.sin(pos * div)
    pe[:, 1::2] = np.cos(pos * div)
    return jnp.asarray(pe)


_PE = _sinusoidal_pe(_SEQ, _D).reshape(-1)

_mesh = plsc.VectorSubcoreMesh(core_axis_name="c", subcore_axis_name="s")


@functools.partial(
    pl.kernel,
    out_type=jax.ShapeDtypeStruct((_ROWS, _D), jnp.float32),
    mesh=_mesh,
    scratch_types=[
        pltpu.VMEM((_NCK, _HC), jnp.int32),
        pltpu.VMEM((_PW * _D,), jnp.float32),
        pltpu.VMEM((_HC, _D), jnp.float32),
        pltpu.VMEM((_HC, _D), jnp.float32),
        pltpu.VMEM((_HC, _D), jnp.float32),
        pltpu.SemaphoreType.DMA,
        pltpu.SemaphoreType.DMA,
        pltpu.SemaphoreType.DMA,
        pltpu.SemaphoreType.DMA,
        pltpu.SemaphoreType.DMA,
        pltpu.SemaphoreType.DMA,
        pltpu.SemaphoreType.DMA,
    ],
)
def _emb_kernel(idx_hbm, table_hbm, pe_hbm, out_hbm,
                idx_v, pe_buf, rows0, rows1, rows2,
                g0, g1, g2, s0, s1, s2, psem):
    wid = lax.axis_index("s") * 2 + lax.axis_index("c")
    pbase = wid * _PW
    bufs = (rows0, rows1, rows2)
    gsems = (g0, g1, g2)
    ssems = (s0, s1, s2)
    pltpu.sync_copy(idx_hbm.at[wid], idx_v)
    gd = [None] * _NCK
    sd = [None] * _NCK
    gd[0] = pltpu.async_copy(table_hbm.at[idx_v.at[0]], bufs[0], gsems[0])
    pe_d = pltpu.async_copy(pe_hbm.at[pl.ds(pbase * _D, _PW * _D)], pe_buf, psem)
    for k in range(_NCK):
        j = k % 3
        if k + 1 < _NCK:
            jn = (k + 1) % 3
            if k - 2 >= 0:
                for d in sd[k - 2]:
                    d.wait()
            gd[k + 1] = pltpu.async_copy(
                table_hbm.at[idx_v.at[k + 1]], bufs[jn], gsems[jn]
            )
        gd[k].wait()
        if k == 0:
            pe_d.wait()

        @plsc.parallel_loop(0, _PC, unroll=2)
        def add_pos(i):
            pe_off = (k * _PC + i) * _D
            for v in range(_VPR):
                pe_vec = pe_buf[pl.ds(pe_off + v * 16, 16)]
                for b in range(_BATCH):
                    plsc.addupdate(
                        bufs[j].at[b * _PC + i, pl.ds(v * 16, 16)], pe_vec
                    )

        sd[k] = [
            pltpu.async_copy(
                bufs[j].at[pl.ds(b * _PC, _PC)],
                out_hbm.at[pl.ds(b * _SEQ + pbase + k * _PC, _PC)],
                ssems[j],
            )
            for b in range(_BATCH)
        ]
    for d in sd[_NCK - 2]:
        d.wait()
    for d in sd[_NCK - 1]:
        d.wait()


def _pe_on_device():
    pos = lax.broadcasted_iota(jnp.float32, (_SEQ, _D // 2), 0)
    two_i = lax.broadcasted_iota(jnp.float32, (_SEQ, _D // 2), 1) * 2.0
    ang = pos * jnp.exp(two_i * (-np.log(10000.0) / _D))
    pe = jnp.stack([jnp.sin(ang), jnp.cos(ang)], axis=-1)
    return pe.reshape(-1)


def kernel(x, table):
    idx = (
        x.reshape(_BATCH, _NW, _NCK, _PC)
        .transpose(1, 2, 0, 3)
        .reshape(_NW, _NCK, _HC)
        .astype(jnp.int32)
    )
    out = _emb_kernel(idx, table, _PE)
    return out.reshape(_BATCH, _SEQ, _D)
